# baseline (speedup 1.0000x reference)
import functools

import jax
import jax.numpy as jnp
from jax import lax
from jax.experimental import pallas as pl
from jax.experimental.pallas import tpu as pltpu
from jax.experimental.pallas import tpu_sc as plsc

D = 1024
B = 8192
NH = 4
HD = D // NH
NE = 4

TG = 2048
TE = 512
NHALF = 2
B2 = B // NHALF
NPAD2 = B2 + NE * TE
NBLK2 = NPAD2 // TE

NC = 2
NS = 16
NW = NC * NS
CH = 16
TPW = B2 // NW
ROWS_W = TPW // CH
NCHUNK = TPW // CH


def _mm(a, w, b=None):
    out = lax.dot_general(a, w, (((1,), (1,)), ((), ())),
                          preferred_element_type=jnp.float32)
    if b is not None:
        out = out + b
    return out


def _gate_kernel(zg_ref, zi_ref, gw_ref, gb_ref,
                 idx_ref, cnt_ref, psum_ref):
    i = pl.program_id(0)
    x = jnp.concatenate([zg_ref[...], zi_ref[...]], axis=1)
    logits = _mm(x, gw_ref[...], gb_ref[...])
    m = jnp.max(logits, axis=1, keepdims=True)
    e = jnp.exp(logits - m)
    probs = e / jnp.sum(e, axis=1, keepdims=True)
    pmax = jnp.max(probs, axis=1, keepdims=True)
    eqf = (probs == pmax).astype(jnp.float32)
    c0, c1, c2 = eqf[:, 0:1], eqf[:, 1:2], eqf[:, 2:3]
    prior = jnp.concatenate(
        [jnp.zeros_like(c0), c0, jnp.maximum(c0, c1),
         jnp.maximum(jnp.maximum(c0, c1), c2)], axis=1)
    onehot = jnp.logical_and(eqf > 0.0, prior == 0.0)
    ohf = onehot.astype(jnp.float32)
    psel = jnp.where(onehot, probs, 0.0)
    lane = lax.broadcasted_iota(jnp.int32, ohf.shape, 1).astype(jnp.float32)
    idx_ref[...] = jnp.sum(ohf * lane, axis=1, keepdims=True).astype(jnp.int32)

    @pl.when(i == 0)
    def _():
        cnt_ref[...] = jnp.zeros_like(cnt_ref)
        psum_ref[...] = jnp.zeros_like(psum_ref)

    cnt_ref[...] += jnp.sum(ohf, axis=0, keepdims=True)
    psum_ref[...] += jnp.sum(psel, axis=0, keepdims=True)


def _fullv(val):
    return jnp.full((16,), val, jnp.int32)


def _sc_hist_kernel(idx2d, hist_out, idxbuf, hbuf):
    wid = lax.axis_index("s") * NC + lax.axis_index("c")
    lanes = lax.iota(jnp.int32, 16)
    zero = jnp.zeros((16,), jnp.int32)
    for k in range(NHALF):
        row0 = k * (B2 // CH) + wid * ROWS_W
        pltpu.sync_copy(idx2d.at[pl.ds(row0, ROWS_W)], idxbuf)
        hv = zero
        for r in range(ROWS_W):
            for h in range(0, CH, 16):
                v = idxbuf[r, pl.ds(h, 16)]
                for e in range(NE):
                    pc = plsc.all_reduce_population_count(v == _fullv(e))
                    hv = hv + jnp.where(lanes == _fullv(e), pc, zero)
        hbuf[...] = hv
        pltpu.sync_copy(hbuf, hist_out.at[k * NW + wid])


def _sc_route_kernel(k, idx2d, curv_all, zg, zi,
                     pos2d, zs_gat, zs_gin,
                     idxbuf, posbuf, curbuf,
                     rowa0, rowa1, rowb0, rowb1,
                     sa0, sa1, sb0, sb1):
    wid = lax.axis_index("s") * NC + lax.axis_index("c")
    row0 = k * (B2 // CH) + wid * ROWS_W
    pltpu.sync_copy(idx2d.at[pl.ds(row0, ROWS_W)], idxbuf)
    pltpu.sync_copy(curv_all.at[wid], curbuf)
    lanes = lax.iota(jnp.int32, 16)
    zero = jnp.zeros((16,), jnp.int32)
    one = jnp.ones((16,), jnp.int32)
    curv = curbuf[...]
    for r in range(ROWS_W):
        for h in range(0, CH, 16):
            v = idxbuf[r, pl.ds(h, 16)]
            curbuf[...] = curv
            basel = plsc.load_gather(curbuf, [v])
            ranks = zero
            for e in range(NE):
                m = v == _fullv(e)
                ci = plsc.cumsum(m.astype(jnp.int32))
                ranks = jnp.where(m, ci - one, ranks)
                pc = plsc.all_reduce_population_count(m)
                curv = curv + jnp.where(lanes == _fullv(e), pc, zero)
            posbuf[r, pl.ds(h, 16)] = basel + ranks
    pltpu.sync_copy(posbuf, pos2d.at[pl.ds(wid * ROWS_W, ROWS_W)])
    rowa = (rowa0, rowa1)
    rowb = (rowb0, rowb1)
    sa = (sa0, sa1)
    sb = (sb0, sb1)
    cpa = [None] * NCHUNK
    cpb = [None] * NCHUNK
    for c in range(NCHUNK):
        p = c % 2
        t0 = k * B2 + wid * TPW + c * CH
        if c >= 2:
            cpa[c - 2].wait()
            cpb[c - 2].wait()
        pltpu.sync_copy(zg.at[pl.ds(t0, CH)], rowa[p])
        cpa[c] = pltpu.async_copy(rowa[p], zs_gat.at[posbuf.at[c]], sa[p])
        pltpu.sync_copy(zi.at[pl.ds(t0, CH)], rowb[p])
        cpb[c] = pltpu.async_copy(rowb[p], zs_gin.at[posbuf.at[c]], sb[p])
    for c in (NCHUNK - 2, NCHUNK - 1):
        cpa[c].wait()
        cpb[c].wait()


def _expert_kernel(be_ref, ap_ref, zg_ref, zi_ref,
                   e0w1_ref, e0b1_ref, e0w2_ref, e0b2_ref,
                   e1w1_ref, e1b1_ref, e1w2_ref, e1b2_ref,
                   wq_ref, bq_ref, wk_ref, bk_ref, wv_ref, bv_ref,
                   e2ow_ref, e2ob_ref, e2fw_ref, e2fb_ref,
                   e3aw_ref, e3ab_ref, e3ow_ref, e3ob_ref,
                   out_ref):
    i = pl.program_id(0)
    be = be_ref[i]

    @pl.when(be == 0)
    def _():
        x = jnp.concatenate([zg_ref[...].astype(jnp.bfloat16),
                             zi_ref[...].astype(jnp.bfloat16)], axis=1)
        h0 = jax.nn.relu(_mm(x, e0w1_ref[...], e0b1_ref[...]))
        out0 = _mm(h0.astype(jnp.bfloat16), e0w2_ref[...], e0b2_ref[...])
        out_ref[...] = ap_ref[0] * out0

    @pl.when(be == 1)
    def _():
        prod = (zg_ref[...] * zi_ref[...]).astype(jnp.bfloat16)
        h1 = jax.nn.relu(_mm(prod, e1w1_ref[...], e1b1_ref[...]))
        out1 = _mm(h1.astype(jnp.bfloat16), e1w2_ref[...], e1b2_ref[...])
        out_ref[...] = ap_ref[1] * out1

    @pl.when(be == 2)
    def _():
        zgb = zg_ref[...].astype(jnp.bfloat16)
        zib = zi_ref[...].astype(jnp.bfloat16)
        q0 = _mm(zgb, wq_ref[...], bq_ref[...])
        q1 = _mm(zib, wq_ref[...], bq_ref[...])
        k0 = _mm(zgb, wk_ref[...], bk_ref[...])
        k1 = _mm(zib, wk_ref[...], bk_ref[...])
        v0 = _mm(zgb, wv_ref[...], bv_ref[...])
        v1 = _mm(zib, wv_ref[...], bv_ref[...])
        scale = 1.0 / (HD ** 0.5)
        ctx_parts = []
        for h in range(NH):
            sl = slice(h * HD, (h + 1) * HD)
            q0h, q1h = q0[:, sl], q1[:, sl]
            k0h, k1h = k0[:, sl], k1[:, sl]
            v0h, v1h = v0[:, sl], v1[:, sl]
            s00 = jnp.sum(q0h * k0h, axis=1, keepdims=True) * scale
            s01 = jnp.sum(q0h * k1h, axis=1, keepdims=True) * scale
            s10 = jnp.sum(q1h * k0h, axis=1, keepdims=True) * scale
            s11 = jnp.sum(q1h * k1h, axis=1, keepdims=True) * scale
            m0 = jnp.maximum(s00, s01)
            a00 = jnp.exp(s00 - m0)
            a01 = jnp.exp(s01 - m0)
            m1 = jnp.maximum(s10, s11)
            a10 = jnp.exp(s10 - m1)
            a11 = jnp.exp(s11 - m1)
            ctx0 = (a00 * v0h + a01 * v1h) / (a00 + a01)
            ctx1 = (a10 * v0h + a11 * v1h) / (a10 + a11)
            ctx_parts.append(0.5 * (ctx0 + ctx1))
        mean_ctx = jnp.concatenate(ctx_parts, axis=1).astype(jnp.bfloat16)
        fused2 = _mm(mean_ctx, e2ow_ref[...], e2ob_ref[...]).astype(jnp.bfloat16)
        out2 = _mm(fused2, e2fw_ref[...], e2fb_ref[...])
        out_ref[...] = ap_ref[2] * out2

    @pl.when(be == 3)
    def _():
        zgf = zg_ref[...]
        zif = zi_ref[...]
        x = jnp.concatenate([zgf.astype(jnp.bfloat16),
                             zif.astype(jnp.bfloat16)], axis=1)
        alpha = jax.nn.sigmoid(_mm(x, e3aw_ref[...], e3ab_ref[...]))
        h3 = (alpha * zgf + (1.0 - alpha) * zif).astype(jnp.bfloat16)
        out3 = _mm(h3, e3ow_ref[...], e3ob_ref[...])
        out_ref[...] = ap_ref[3] * out3


def _sc_combine_kernel(outs0, outs1, pos0, pos1, out,
                       posbuf, row0, row1, s0, s1):
    wid = lax.axis_index("s") * NC + lax.axis_index("c")
    row = (row0, row1)
    sem = (s0, s1)
    for k in range(NHALF):
        outs = (outs0, outs1)[k]
        pos = (pos0, pos1)[k]
        pltpu.sync_copy(pos.at[pl.ds(wid * ROWS_W, ROWS_W)], posbuf)
        cps = [None] * NCHUNK
        cps[0] = pltpu.async_copy(outs.at[posbuf.at[0]], row[0], sem[0])
        for c in range(1, NCHUNK + 1):
            if c < NCHUNK:
                cps[c] = pltpu.async_copy(outs.at[posbuf.at[c]], row[c % 2],
                                          sem[c % 2])
            cps[c - 1].wait()
            t0 = k * B2 + wid * TPW + (c - 1) * CH
            pltpu.sync_copy(row[(c - 1) % 2], out.at[pl.ds(t0, CH)])


@jax.jit
def kernel(z_gat, z_gin, gate_W, gate_b,
           e0_fc1_W, e0_fc1_b, e0_fc2_W, e0_fc2_b,
           e1_fc1_W, e1_fc1_b, e1_fc2_W, e1_fc2_b,
           e2_in_W, e2_in_b, e2_out_W, e2_out_b, e2_fc_W, e2_fc_b,
           e3_alpha_W, e3_alpha_b, e3_out_W, e3_out_b):
    f32 = jnp.float32
    i32 = jnp.int32
    bf16 = jnp.bfloat16

    grid_g = B // TG
    idx, cnt, psum = pl.pallas_call(
        _gate_kernel,
        grid=(grid_g,),
        in_specs=[
            pl.BlockSpec((TG, D), lambda i: (i, 0)),
            pl.BlockSpec((TG, D), lambda i: (i, 0)),
            pl.BlockSpec((4, 2 * D), lambda i: (0, 0)),
            pl.BlockSpec((1, 4), lambda i: (0, 0)),
        ],
        out_specs=[
            pl.BlockSpec((TG, 1), lambda i: (i, 0)),
            pl.BlockSpec((1, 4), lambda i: (0, 0)),
            pl.BlockSpec((1, 4), lambda i: (0, 0)),
        ],
        out_shape=[
            jax.ShapeDtypeStruct((B, 1), i32),
            jax.ShapeDtypeStruct((1, 4), f32),
            jax.ShapeDtypeStruct((1, 4), f32),
        ],
    )(z_gat, z_gin, gate_W, gate_b.reshape(1, 4))

    counts = cnt[0]
    avg_prob = jnp.where(counts > 0, psum[0] / jnp.maximum(counts, 1.0), 0.0)
    aux_loss = jnp.sum((counts / float(B)) ** 2) * 4.0

    idx2d = idx.reshape(B // CH, CH)

    mesh = plsc.VectorSubcoreMesh(core_axis_name="c", subcore_axis_name="s",
                                  num_cores=NC, num_subcores=NS)
    sc_params = pltpu.CompilerParams(needs_layout_passes=False)

    hist = pl.kernel(
        _sc_hist_kernel,
        out_type=jax.ShapeDtypeStruct((NHALF * NW, 16), i32),
        mesh=mesh,
        compiler_params=sc_params,
        scratch_types=[
            pltpu.VMEM((ROWS_W, CH), i32),
            pltpu.VMEM((16,), i32),
        ],
    )(idx2d)

    wq, wk, wv = jnp.split(e2_in_W, 3, axis=0)
    bq, bk, bv = jnp.split(e2_in_b, 3, axis=0)

    def wcast(w):
        return w.astype(bf16)

    def b2d(b):
        return b.reshape(1, -1).astype(f32)

    weight_args = (
        wcast(e0_fc1_W), b2d(e0_fc1_b), wcast(e0_fc2_W), b2d(e0_fc2_b),
        wcast(e1_fc1_W), b2d(e1_fc1_b), wcast(e1_fc2_W), b2d(e1_fc2_b),
        wcast(wq), b2d(bq), wcast(wk), b2d(bk), wcast(wv), b2d(bv),
        wcast(e2_out_W), b2d(e2_out_b), wcast(e2_fc_W), b2d(e2_fc_b),
        wcast(e3_alpha_W), b2d(e3_alpha_b), wcast(e3_out_W), b2d(e3_out_b),
    )

    def wspec(w):
        return pl.BlockSpec(w.shape, lambda i: tuple(0 for _ in w.shape))

    outs = []
    poss = []
    for k in range(NHALF):
        hist_k = lax.slice(hist, (k * NW, 0), ((k + 1) * NW, 16))
        cnt_k = jnp.sum(hist_k, axis=0)
        pcnt_k = jnp.bitwise_and(cnt_k + (TE - 1), jnp.int32(-TE))
        seg_k = jnp.cumsum(pcnt_k) - pcnt_k
        curv_all_k = seg_k[None, :] + (jnp.cumsum(hist_k, axis=0) - hist_k)

        bs = jnp.arange(NBLK2, dtype=i32) * TE
        block_expert = jnp.full((NBLK2,), -1, i32)
        for e in range(NE):
            in_seg = (bs >= seg_k[e]) & (bs < seg_k[e] + pcnt_k[e])
            block_expert = jnp.where(in_seg, e, block_expert)

        pos2d_k, zs_gat, zs_gin = pl.kernel(
            functools.partial(_sc_route_kernel, k),
            out_type=[
                jax.ShapeDtypeStruct((B2 // CH, CH), i32),
                jax.ShapeDtypeStruct((NPAD2, D), f32),
                jax.ShapeDtypeStruct((NPAD2, D), f32),
            ],
            mesh=mesh,
            compiler_params=sc_params,
            scratch_types=[
                pltpu.VMEM((ROWS_W, CH), i32),
                pltpu.VMEM((ROWS_W, CH), i32),
                pltpu.VMEM((16,), i32),
                pltpu.VMEM((CH, D), f32),
                pltpu.VMEM((CH, D), f32),
                pltpu.VMEM((CH, D), f32),
                pltpu.VMEM((CH, D), f32),
                pltpu.SemaphoreType.DMA,
                pltpu.SemaphoreType.DMA,
                pltpu.SemaphoreType.DMA,
                pltpu.SemaphoreType.DMA,
            ],
        )(idx2d, curv_all_k, z_gat, z_gin)

        out_sorted = pl.pallas_call(
            _expert_kernel,
            grid=(NBLK2,),
            in_specs=[
                pl.BlockSpec(memory_space=pltpu.SMEM),
                pl.BlockSpec(memory_space=pltpu.SMEM),
                pl.BlockSpec((TE, D), lambda i: (i, 0)),
                pl.BlockSpec((TE, D), lambda i: (i, 0)),
            ] + [wspec(w) for w in weight_args],
            out_specs=pl.BlockSpec((TE, D), lambda i: (i, 0)),
            out_shape=jax.ShapeDtypeStruct((NPAD2, D), f32),
        )(block_expert, avg_prob, zs_gat, zs_gin, *weight_args)

        outs.append(out_sorted)
        poss.append(pos2d_k)

    output = pl.kernel(
        _sc_combine_kernel,
        out_type=jax.ShapeDtypeStruct((B, D), f32),
        mesh=mesh,
        compiler_params=sc_params,
        scratch_types=[
            pltpu.VMEM((ROWS_W, CH), i32),
            pltpu.VMEM((CH, D), f32),
            pltpu.VMEM((CH, D), f32),
            pltpu.SemaphoreType.DMA,
            pltpu.SemaphoreType.DMA,
        ],
    )(outs[0], outs[1], poss[0], poss[1])

    return output, aux_loss


# speedup vs baseline: 1.1515x; 1.1515x over previous
import functools

import jax
import jax.numpy as jnp
from jax import lax
from jax.experimental import pallas as pl
from jax.experimental.pallas import tpu as pltpu
from jax.experimental.pallas import tpu_sc as plsc

D = 1024
B = 8192
NH = 4
HD = D // NH
NE = 4

TG = 2048
TE = 512
NPAD = B + NE * TE
NBLK = NPAD // TE

NC = 2
NS = 16
NW = NC * NS
TPW = B // NW
CH = 16
NCHUNK = TPW // CH
ROWS_W = TPW // CH


def _mm(a, w, b=None):
    out = lax.dot_general(a, w, (((1,), (1,)), ((), ())),
                          preferred_element_type=jnp.float32)
    if b is not None:
        out = out + b
    return out


def _gate_kernel(zg_ref, zi_ref, gw_ref, gb_ref,
                 p_ref, idx_ref, cnt_ref, psum_ref):
    i = pl.program_id(0)
    x = jnp.concatenate([zg_ref[...], zi_ref[...]], axis=1)
    logits = _mm(x, gw_ref[...], gb_ref[...])
    m = jnp.max(logits, axis=1, keepdims=True)
    e = jnp.exp(logits - m)
    probs = e / jnp.sum(e, axis=1, keepdims=True)
    pmax = jnp.max(probs, axis=1, keepdims=True)
    eqf = (probs == pmax).astype(jnp.float32)
    c0, c1, c2 = eqf[:, 0:1], eqf[:, 1:2], eqf[:, 2:3]
    prior = jnp.concatenate(
        [jnp.zeros_like(c0), c0, jnp.maximum(c0, c1),
         jnp.maximum(jnp.maximum(c0, c1), c2)], axis=1)
    onehot = jnp.logical_and(eqf > 0.0, prior == 0.0)
    ohf = onehot.astype(jnp.float32)
    psel = jnp.where(onehot, probs, 0.0)
    p_ref[...] = psel
    lane = lax.broadcasted_iota(jnp.int32, ohf.shape, 1).astype(jnp.float32)
    idx_ref[...] = jnp.sum(ohf * lane, axis=1, keepdims=True).astype(jnp.int32)

    @pl.when(i == 0)
    def _():
        cnt_ref[...] = jnp.zeros_like(cnt_ref)
        psum_ref[...] = jnp.zeros_like(psum_ref)

    cnt_ref[...] += jnp.sum(ohf, axis=0, keepdims=True)
    psum_ref[...] += jnp.sum(psel, axis=0, keepdims=True)


def _fullv(val):
    return jnp.full((16,), val, jnp.int32)


def _sc_hist_kernel(idx2d, hist_out, idxbuf, hbuf):
    wid = lax.axis_index("s") * NC + lax.axis_index("c")
    pltpu.sync_copy(idx2d.at[pl.ds(wid * ROWS_W, ROWS_W)], idxbuf)
    lanes = lax.iota(jnp.int32, 16)
    zero = jnp.zeros((16,), jnp.int32)
    hv = zero
    for r in range(ROWS_W):
        for h in range(0, CH, 16):
            v = idxbuf[r, pl.ds(h, 16)]
            for e in range(NE):
                pc = plsc.all_reduce_population_count(v == _fullv(e))
                hv = hv + jnp.where(lanes == _fullv(e), pc, zero)
    hbuf[...] = hv
    pltpu.sync_copy(hbuf, hist_out.at[wid])


def _sc_route_kernel(idx2d, curv_all, zg, zi,
                     pos2d, zs,
                     idxbuf, posbuf, curbuf,
                     rowab0, rowab1, sa0, sa1):
    wid = lax.axis_index("s") * NC + lax.axis_index("c")
    pltpu.sync_copy(idx2d.at[pl.ds(wid * ROWS_W, ROWS_W)], idxbuf)
    pltpu.sync_copy(curv_all.at[wid], curbuf)
    lanes = lax.iota(jnp.int32, 16)
    zero = jnp.zeros((16,), jnp.int32)
    one = jnp.ones((16,), jnp.int32)
    curv = curbuf[...]
    for r in range(ROWS_W):
        for h in range(0, CH, 16):
            v = idxbuf[r, pl.ds(h, 16)]
            curbuf[...] = curv
            basel = plsc.load_gather(curbuf, [v])
            ranks = zero
            for e in range(NE):
                m = v == _fullv(e)
                ci = plsc.cumsum(m.astype(jnp.int32))
                ranks = jnp.where(m, ci - one, ranks)
                pc = plsc.all_reduce_population_count(m)
                curv = curv + jnp.where(lanes == _fullv(e), pc, zero)
            posbuf[r, pl.ds(h, 16)] = basel + ranks
    pltpu.sync_copy(posbuf, pos2d.at[pl.ds(wid * ROWS_W, ROWS_W)])
    rowab = (rowab0, rowab1)
    sa = (sa0, sa1)
    cpa = [None] * NCHUNK
    for c in range(NCHUNK):
        p = c % 2
        t0 = wid * TPW + c * CH
        if c >= 2:
            cpa[c - 2].wait()
        pltpu.sync_copy(zg.at[pl.ds(t0, CH)], rowab[p].at[:, pl.ds(0, D)])
        pltpu.sync_copy(zi.at[pl.ds(t0, CH)], rowab[p].at[:, pl.ds(D, D)])
        cpa[c] = pltpu.async_copy(rowab[p], zs.at[posbuf.at[c]], sa[p])
    for c in (NCHUNK - 2, NCHUNK - 1):
        cpa[c].wait()


def _expert_kernel(be_ref, ap_ref, zs_ref,
                   e0w1_ref, e0b1_ref, e0w2_ref, e0b2_ref,
                   e1w1_ref, e1b1_ref, e1w2_ref, e1b2_ref,
                   wq_ref, bq_ref, wk_ref, bk_ref, wv_ref, bv_ref,
                   e2ow_ref, e2ob_ref, e2fw_ref, e2fb_ref,
                   e3aw_ref, e3ab_ref, e3ow_ref, e3ob_ref,
                   out_ref):
    i = pl.program_id(0)
    be = be_ref[i]

    @pl.when(be == 0)
    def _():
        x = zs_ref[...].astype(jnp.bfloat16)
        h0 = jax.nn.relu(_mm(x, e0w1_ref[...], e0b1_ref[...]))
        out0 = _mm(h0.astype(jnp.bfloat16), e0w2_ref[...], e0b2_ref[...])
        out_ref[...] = ap_ref[0] * out0

    @pl.when(be == 1)
    def _():
        prod = (zs_ref[:, :D] * zs_ref[:, D:]).astype(jnp.bfloat16)
        h1 = jax.nn.relu(_mm(prod, e1w1_ref[...], e1b1_ref[...]))
        out1 = _mm(h1.astype(jnp.bfloat16), e1w2_ref[...], e1b2_ref[...])
        out_ref[...] = ap_ref[1] * out1

    @pl.when(be == 2)
    def _():
        zgb = zs_ref[:, :D].astype(jnp.bfloat16)
        zib = zs_ref[:, D:].astype(jnp.bfloat16)
        q0 = _mm(zgb, wq_ref[...], bq_ref[...])
        q1 = _mm(zib, wq_ref[...], bq_ref[...])
        k0 = _mm(zgb, wk_ref[...], bk_ref[...])
        k1 = _mm(zib, wk_ref[...], bk_ref[...])
        v0 = _mm(zgb, wv_ref[...], bv_ref[...])
        v1 = _mm(zib, wv_ref[...], bv_ref[...])
        scale = 1.0 / (HD ** 0.5)
        ctx_parts = []
        for h in range(NH):
            sl = slice(h * HD, (h + 1) * HD)
            q0h, q1h = q0[:, sl], q1[:, sl]
            k0h, k1h = k0[:, sl], k1[:, sl]
            v0h, v1h = v0[:, sl], v1[:, sl]
            s00 = jnp.sum(q0h * k0h, axis=1, keepdims=True) * scale
            s01 = jnp.sum(q0h * k1h, axis=1, keepdims=True) * scale
            s10 = jnp.sum(q1h * k0h, axis=1, keepdims=True) * scale
            s11 = jnp.sum(q1h * k1h, axis=1, keepdims=True) * scale
            m0 = jnp.maximum(s00, s01)
            a00 = jnp.exp(s00 - m0)
            a01 = jnp.exp(s01 - m0)
            m1 = jnp.maximum(s10, s11)
            a10 = jnp.exp(s10 - m1)
            a11 = jnp.exp(s11 - m1)
            ctx0 = (a00 * v0h + a01 * v1h) / (a00 + a01)
            ctx1 = (a10 * v0h + a11 * v1h) / (a10 + a11)
            ctx_parts.append(0.5 * (ctx0 + ctx1))
        mean_ctx = jnp.concatenate(ctx_parts, axis=1).astype(jnp.bfloat16)
        fused2 = _mm(mean_ctx, e2ow_ref[...], e2ob_ref[...]).astype(jnp.bfloat16)
        out2 = _mm(fused2, e2fw_ref[...], e2fb_ref[...])
        out_ref[...] = ap_ref[2] * out2

    @pl.when(be == 3)
    def _():
        zgf = zs_ref[:, :D]
        zif = zs_ref[:, D:]
        x = zs_ref[...].astype(jnp.bfloat16)
        alpha = jax.nn.sigmoid(_mm(x, e3aw_ref[...], e3ab_ref[...]))
        h3 = (alpha * zgf + (1.0 - alpha) * zif).astype(jnp.bfloat16)
        out3 = _mm(h3, e3ow_ref[...], e3ob_ref[...])
        out_ref[...] = ap_ref[3] * out3


def _sc_combine_kernel(outs, pos2d, out, posbuf, row0, row1, s0, s1):
    wid = lax.axis_index("s") * NC + lax.axis_index("c")
    pltpu.sync_copy(pos2d.at[pl.ds(wid * ROWS_W, ROWS_W)], posbuf)
    row = (row0, row1)
    sem = (s0, s1)
    cps = [None] * NCHUNK
    cps[0] = pltpu.async_copy(outs.at[posbuf.at[0]], row[0], sem[0])
    for c in range(1, NCHUNK + 1):
        if c < NCHUNK:
            cps[c] = pltpu.async_copy(outs.at[posbuf.at[c]], row[c % 2],
                                      sem[c % 2])
        cps[c - 1].wait()
        t0 = wid * TPW + (c - 1) * CH
        pltpu.sync_copy(row[(c - 1) % 2], out.at[pl.ds(t0, CH)])


@jax.jit
def kernel(z_gat, z_gin, gate_W, gate_b,
           e0_fc1_W, e0_fc1_b, e0_fc2_W, e0_fc2_b,
           e1_fc1_W, e1_fc1_b, e1_fc2_W, e1_fc2_b,
           e2_in_W, e2_in_b, e2_out_W, e2_out_b, e2_fc_W, e2_fc_b,
           e3_alpha_W, e3_alpha_b, e3_out_W, e3_out_b):
    f32 = jnp.float32
    i32 = jnp.int32
    bf16 = jnp.bfloat16

    grid_g = B // TG
    _, idx, cnt, psum = pl.pallas_call(
        _gate_kernel,
        grid=(grid_g,),
        in_specs=[
            pl.BlockSpec((TG, D), lambda i: (i, 0)),
            pl.BlockSpec((TG, D), lambda i: (i, 0)),
            pl.BlockSpec((4, 2 * D), lambda i: (0, 0)),
            pl.BlockSpec((1, 4), lambda i: (0, 0)),
        ],
        out_specs=[
            pl.BlockSpec((TG, 4), lambda i: (i, 0)),
            pl.BlockSpec((TG, 1), lambda i: (i, 0)),
            pl.BlockSpec((1, 4), lambda i: (0, 0)),
            pl.BlockSpec((1, 4), lambda i: (0, 0)),
        ],
        out_shape=[
            jax.ShapeDtypeStruct((B, 4), f32),
            jax.ShapeDtypeStruct((B, 1), i32),
            jax.ShapeDtypeStruct((1, 4), f32),
            jax.ShapeDtypeStruct((1, 4), f32),
        ],
    )(z_gat, z_gin, gate_W, gate_b.reshape(1, 4))

    counts = cnt[0]
    avg_prob = jnp.where(counts > 0, psum[0] / jnp.maximum(counts, 1.0), 0.0)
    aux_loss = jnp.sum((counts / float(B)) ** 2) * 4.0

    cnt_i = counts.astype(i32)
    pcnt = jnp.bitwise_and(cnt_i + (TE - 1), jnp.int32(-TE))
    seg = jnp.cumsum(pcnt) - pcnt
    bs = jnp.arange(NBLK, dtype=i32) * TE
    block_expert = jnp.full((NBLK,), -1, i32)
    for e in range(NE):
        in_seg = (bs >= seg[e]) & (bs < seg[e] + pcnt[e])
        block_expert = jnp.where(in_seg, e, block_expert)

    idx2d = idx.reshape(B // CH, CH)

    mesh = plsc.VectorSubcoreMesh(core_axis_name="c", subcore_axis_name="s",
                                  num_cores=NC, num_subcores=NS)

    hist = pl.kernel(
        _sc_hist_kernel,
        out_type=jax.ShapeDtypeStruct((NW, 16), i32),
        mesh=mesh,
        compiler_params=pltpu.CompilerParams(needs_layout_passes=False),
        scratch_types=[
            pltpu.VMEM((ROWS_W, CH), i32),
            pltpu.VMEM((16,), i32),
        ],
    )(idx2d)

    seg16 = jnp.zeros((16,), i32).at[:NE].set(seg)
    curv_all = seg16[None, :] + (jnp.cumsum(hist, axis=0) - hist)

    pos2d, zs = pl.kernel(
        _sc_route_kernel,
        out_type=[
            jax.ShapeDtypeStruct((B // CH, CH), i32),
            jax.ShapeDtypeStruct((NPAD, 2 * D), f32),
        ],
        mesh=mesh,
        compiler_params=pltpu.CompilerParams(needs_layout_passes=False),
        scratch_types=[
            pltpu.VMEM((ROWS_W, CH), i32),
            pltpu.VMEM((ROWS_W, CH), i32),
            pltpu.VMEM((16,), i32),
            pltpu.VMEM((CH, 2 * D), f32),
            pltpu.VMEM((CH, 2 * D), f32),
            pltpu.SemaphoreType.DMA,
            pltpu.SemaphoreType.DMA,
        ],
    )(idx2d, curv_all, z_gat, z_gin)

    wq, wk, wv = jnp.split(e2_in_W, 3, axis=0)
    bq, bk, bv = jnp.split(e2_in_b, 3, axis=0)

    def wcast(w):
        return w.astype(bf16)

    def b2d(b):
        return b.reshape(1, -1).astype(f32)

    weight_args = (
        wcast(e0_fc1_W), b2d(e0_fc1_b), wcast(e0_fc2_W), b2d(e0_fc2_b),
        wcast(e1_fc1_W), b2d(e1_fc1_b), wcast(e1_fc2_W), b2d(e1_fc2_b),
        wcast(wq), b2d(bq), wcast(wk), b2d(bk), wcast(wv), b2d(bv),
        wcast(e2_out_W), b2d(e2_out_b), wcast(e2_fc_W), b2d(e2_fc_b),
        wcast(e3_alpha_W), b2d(e3_alpha_b), wcast(e3_out_W), b2d(e3_out_b),
    )

    def wspec(w):
        return pl.BlockSpec(w.shape, lambda i: tuple(0 for _ in w.shape))

    out_sorted = pl.pallas_call(
        _expert_kernel,
        grid=(NBLK,),
        in_specs=[
            pl.BlockSpec(memory_space=pltpu.SMEM),
            pl.BlockSpec(memory_space=pltpu.SMEM),
            pl.BlockSpec((TE, 2 * D), lambda i: (i, 0)),
        ] + [wspec(w) for w in weight_args],
        out_specs=pl.BlockSpec((TE, D), lambda i: (i, 0)),
        out_shape=jax.ShapeDtypeStruct((NPAD, D), f32),
    )(block_expert, avg_prob, zs, *weight_args)

    output = pl.kernel(
        _sc_combine_kernel,
        out_type=jax.ShapeDtypeStruct((B, D), f32),
        mesh=mesh,
        compiler_params=pltpu.CompilerParams(needs_layout_passes=False),
        scratch_types=[
            pltpu.VMEM((ROWS_W, CH), i32),
            pltpu.VMEM((CH, D), f32),
            pltpu.VMEM((CH, D), f32),
            pltpu.SemaphoreType.DMA,
            pltpu.SemaphoreType.DMA,
        ],
    )(out_sorted, pos2d)

    return output, aux_loss


# speedup vs baseline: 1.1617x; 1.0088x over previous
import functools

import jax
import jax.numpy as jnp
from jax import lax
from jax.experimental import pallas as pl
from jax.experimental.pallas import tpu as pltpu
from jax.experimental.pallas import tpu_sc as plsc

D = 1024
B = 8192
NH = 4
HD = D // NH
NE = 4

TG = 2048
TE = 512
NPAD = B + NE * TE
NBLK = NPAD // TE

NC = 2
NS = 16
NW = NC * NS
TPW = B // NW
CH = 16
NCHUNK = TPW // CH
ROWS_W = TPW // CH


def _mm(a, w, b=None):
    out = lax.dot_general(a, w, (((1,), (1,)), ((), ())),
                          preferred_element_type=jnp.float32)
    if b is not None:
        out = out + b
    return out


def _gate_kernel(zg_ref, zi_ref, gw_ref, gb_ref,
                 idx_ref, cnt_ref, psum_ref, hist_ref):
    i = pl.program_id(0)
    x = jnp.concatenate([zg_ref[...], zi_ref[...]], axis=1)
    logits = _mm(x, gw_ref[...], gb_ref[...])
    m = jnp.max(logits, axis=1, keepdims=True)
    e = jnp.exp(logits - m)
    probs = e / jnp.sum(e, axis=1, keepdims=True)
    pmax = jnp.max(probs, axis=1, keepdims=True)
    eqf = (probs == pmax).astype(jnp.float32)
    c0, c1, c2 = eqf[:, 0:1], eqf[:, 1:2], eqf[:, 2:3]
    prior = jnp.concatenate(
        [jnp.zeros_like(c0), c0, jnp.maximum(c0, c1),
         jnp.maximum(jnp.maximum(c0, c1), c2)], axis=1)
    onehot = jnp.logical_and(eqf > 0.0, prior == 0.0)
    ohf = onehot.astype(jnp.float32)
    psel = jnp.where(onehot, probs, 0.0)
    lane = lax.broadcasted_iota(jnp.int32, ohf.shape, 1).astype(jnp.float32)
    idx_ref[...] = jnp.sum(ohf * lane, axis=1, keepdims=True).astype(jnp.int32)
    hist_ref[...] = jnp.concatenate(
        [jnp.sum(ohf[w * TPW:(w + 1) * TPW], axis=0, keepdims=True)
         for w in range(TG // TPW)], axis=0)

    @pl.when(i == 0)
    def _():
        cnt_ref[...] = jnp.zeros_like(cnt_ref)
        psum_ref[...] = jnp.zeros_like(psum_ref)

    cnt_ref[...] += jnp.sum(ohf, axis=0, keepdims=True)
    psum_ref[...] += jnp.sum(psel, axis=0, keepdims=True)


def _fullv(val):
    return jnp.full((16,), val, jnp.int32)


def _sc_route_kernel(idx2d, curv_all, zg, zi,
                     pos2d, zs,
                     idxbuf, posbuf, curbuf,
                     rowab0, rowab1, sa0, sa1):
    wid = lax.axis_index("s") * NC + lax.axis_index("c")
    pltpu.sync_copy(idx2d.at[pl.ds(wid * ROWS_W, ROWS_W)], idxbuf)
    pltpu.sync_copy(curv_all.at[wid], curbuf)
    lanes = lax.iota(jnp.int32, 16)
    zero = jnp.zeros((16,), jnp.int32)
    one = jnp.ones((16,), jnp.int32)
    curv = curbuf[...]
    for r in range(ROWS_W):
        for h in range(0, CH, 16):
            v = idxbuf[r, pl.ds(h, 16)]
            curbuf[...] = curv
            basel = plsc.load_gather(curbuf, [v])
            ranks = zero
            for e in range(NE):
                m = v == _fullv(e)
                ci = plsc.cumsum(m.astype(jnp.int32))
                ranks = jnp.where(m, ci - one, ranks)
                pc = plsc.all_reduce_population_count(m)
                curv = curv + jnp.where(lanes == _fullv(e), pc, zero)
            posbuf[r, pl.ds(h, 16)] = basel + ranks
    pltpu.sync_copy(posbuf, pos2d.at[pl.ds(wid * ROWS_W, ROWS_W)])
    rowab = (rowab0, rowab1)
    sa = (sa0, sa1)
    cpa = [None] * NCHUNK
    for c in range(NCHUNK):
        p = c % 2
        t0 = wid * TPW + c * CH
        if c >= 2:
            cpa[c - 2].wait()
        pltpu.sync_copy(zg.at[pl.ds(t0, CH)], rowab[p].at[:, pl.ds(0, D)])
        pltpu.sync_copy(zi.at[pl.ds(t0, CH)], rowab[p].at[:, pl.ds(D, D)])
        cpa[c] = pltpu.async_copy(rowab[p], zs.at[posbuf.at[c]], sa[p])
    for c in (NCHUNK - 2, NCHUNK - 1):
        cpa[c].wait()


def _expert_kernel(be_ref, ap_ref, zs_ref,
                   e0w1_ref, e0b1_ref, e0w2_ref, e0b2_ref,
                   e1w1_ref, e1b1_ref, e1w2_ref, e1b2_ref,
                   wq_ref, bq_ref, wk_ref, bk_ref, wv_ref, bv_ref,
                   e2ow_ref, e2ob_ref, e2fw_ref, e2fb_ref,
                   e3aw_ref, e3ab_ref, e3ow_ref, e3ob_ref,
                   out_ref):
    i = pl.program_id(0)
    be = be_ref[i]

    @pl.when(be == 0)
    def _():
        x = zs_ref[...].astype(jnp.bfloat16)
        h0 = jax.nn.relu(_mm(x, e0w1_ref[...], e0b1_ref[...]))
        out0 = _mm(h0.astype(jnp.bfloat16), e0w2_ref[...], e0b2_ref[...])
        out_ref[...] = ap_ref[0] * out0

    @pl.when(be == 1)
    def _():
        prod = (zs_ref[:, :D] * zs_ref[:, D:]).astype(jnp.bfloat16)
        h1 = jax.nn.relu(_mm(prod, e1w1_ref[...], e1b1_ref[...]))
        out1 = _mm(h1.astype(jnp.bfloat16), e1w2_ref[...], e1b2_ref[...])
        out_ref[...] = ap_ref[1] * out1

    @pl.when(be == 2)
    def _():
        zgb = zs_ref[:, :D].astype(jnp.bfloat16)
        zib = zs_ref[:, D:].astype(jnp.bfloat16)
        q0 = _mm(zgb, wq_ref[...], bq_ref[...])
        q1 = _mm(zib, wq_ref[...], bq_ref[...])
        k0 = _mm(zgb, wk_ref[...], bk_ref[...])
        k1 = _mm(zib, wk_ref[...], bk_ref[...])
        v0 = _mm(zgb, wv_ref[...], bv_ref[...])
        v1 = _mm(zib, wv_ref[...], bv_ref[...])
        scale = 1.0 / (HD ** 0.5)
        ctx_parts = []
        for h in range(NH):
            sl = slice(h * HD, (h + 1) * HD)
            q0h, q1h = q0[:, sl], q1[:, sl]
            k0h, k1h = k0[:, sl], k1[:, sl]
            v0h, v1h = v0[:, sl], v1[:, sl]
            s00 = jnp.sum(q0h * k0h, axis=1, keepdims=True) * scale
            s01 = jnp.sum(q0h * k1h, axis=1, keepdims=True) * scale
            s10 = jnp.sum(q1h * k0h, axis=1, keepdims=True) * scale
            s11 = jnp.sum(q1h * k1h, axis=1, keepdims=True) * scale
            m0 = jnp.maximum(s00, s01)
            a00 = jnp.exp(s00 - m0)
            a01 = jnp.exp(s01 - m0)
            m1 = jnp.maximum(s10, s11)
            a10 = jnp.exp(s10 - m1)
            a11 = jnp.exp(s11 - m1)
            ctx0 = (a00 * v0h + a01 * v1h) / (a00 + a01)
            ctx1 = (a10 * v0h + a11 * v1h) / (a10 + a11)
            ctx_parts.append(0.5 * (ctx0 + ctx1))
        mean_ctx = jnp.concatenate(ctx_parts, axis=1).astype(jnp.bfloat16)
        fused2 = _mm(mean_ctx, e2ow_ref[...], e2ob_ref[...]).astype(jnp.bfloat16)
        out2 = _mm(fused2, e2fw_ref[...], e2fb_ref[...])
        out_ref[...] = ap_ref[2] * out2

    @pl.when(be == 3)
    def _():
        zgf = zs_ref[:, :D]
        zif = zs_ref[:, D:]
        x = zs_ref[...].astype(jnp.bfloat16)
        alpha = jax.nn.sigmoid(_mm(x, e3aw_ref[...], e3ab_ref[...]))
        h3 = (alpha * zgf + (1.0 - alpha) * zif).astype(jnp.bfloat16)
        out3 = _mm(h3, e3ow_ref[...], e3ob_ref[...])
        out_ref[...] = ap_ref[3] * out3


def _sc_combine_kernel(outs, pos2d, out, posbuf, row0, row1, s0, s1):
    wid = lax.axis_index("s") * NC + lax.axis_index("c")
    pltpu.sync_copy(pos2d.at[pl.ds(wid * ROWS_W, ROWS_W)], posbuf)
    row = (row0, row1)
    sem = (s0, s1)
    cps = [None] * NCHUNK
    cps[0] = pltpu.async_copy(outs.at[posbuf.at[0]], row[0], sem[0])
    for c in range(1, NCHUNK + 1):
        if c < NCHUNK:
            cps[c] = pltpu.async_copy(outs.at[posbuf.at[c]], row[c % 2],
                                      sem[c % 2])
        cps[c - 1].wait()
        t0 = wid * TPW + (c - 1) * CH
        pltpu.sync_copy(row[(c - 1) % 2], out.at[pl.ds(t0, CH)])


@jax.jit
def kernel(z_gat, z_gin, gate_W, gate_b,
           e0_fc1_W, e0_fc1_b, e0_fc2_W, e0_fc2_b,
           e1_fc1_W, e1_fc1_b, e1_fc2_W, e1_fc2_b,
           e2_in_W, e2_in_b, e2_out_W, e2_out_b, e2_fc_W, e2_fc_b,
           e3_alpha_W, e3_alpha_b, e3_out_W, e3_out_b):
    f32 = jnp.float32
    i32 = jnp.int32
    bf16 = jnp.bfloat16

    grid_g = B // TG
    idx, cnt, psum, hist4 = pl.pallas_call(
        _gate_kernel,
        grid=(grid_g,),
        in_specs=[
            pl.BlockSpec((TG, D), lambda i: (i, 0)),
            pl.BlockSpec((TG, D), lambda i: (i, 0)),
            pl.BlockSpec((4, 2 * D), lambda i: (0, 0)),
            pl.BlockSpec((1, 4), lambda i: (0, 0)),
        ],
        out_specs=[
            pl.BlockSpec((TG, 1), lambda i: (i, 0)),
            pl.BlockSpec((1, 4), lambda i: (0, 0)),
            pl.BlockSpec((1, 4), lambda i: (0, 0)),
            pl.BlockSpec((TG // TPW, 4), lambda i: (i, 0)),
        ],
        out_shape=[
            jax.ShapeDtypeStruct((B, 1), i32),
            jax.ShapeDtypeStruct((1, 4), f32),
            jax.ShapeDtypeStruct((1, 4), f32),
            jax.ShapeDtypeStruct((NW, 4), f32),
        ],
    )(z_gat, z_gin, gate_W, gate_b.reshape(1, 4))

    counts = cnt[0]
    avg_prob = jnp.where(counts > 0, psum[0] / jnp.maximum(counts, 1.0), 0.0)
    aux_loss = jnp.sum((counts / float(B)) ** 2) * 4.0

    cnt_i = counts.astype(i32)
    pcnt = jnp.bitwise_and(cnt_i + (TE - 1), jnp.int32(-TE))
    seg = jnp.cumsum(pcnt) - pcnt
    bs = jnp.arange(NBLK, dtype=i32) * TE
    block_expert = jnp.full((NBLK,), -1, i32)
    for e in range(NE):
        in_seg = (bs >= seg[e]) & (bs < seg[e] + pcnt[e])
        block_expert = jnp.where(in_seg, e, block_expert)

    idx2d = idx.reshape(B // CH, CH)

    mesh = plsc.VectorSubcoreMesh(core_axis_name="c", subcore_axis_name="s",
                                  num_cores=NC, num_subcores=NS)

    hist = jnp.zeros((NW, 16), i32).at[:, :NE].set(hist4.astype(i32))
    seg16 = jnp.zeros((16,), i32).at[:NE].set(seg)
    curv_all = seg16[None, :] + (jnp.cumsum(hist, axis=0) - hist)

    pos2d, zs = pl.kernel(
        _sc_route_kernel,
        out_type=[
            jax.ShapeDtypeStruct((B // CH, CH), i32),
            jax.ShapeDtypeStruct((NPAD, 2 * D), f32),
        ],
        mesh=mesh,
        compiler_params=pltpu.CompilerParams(needs_layout_passes=False),
        scratch_types=[
            pltpu.VMEM((ROWS_W, CH), i32),
            pltpu.VMEM((ROWS_W, CH), i32),
            pltpu.VMEM((16,), i32),
            pltpu.VMEM((CH, 2 * D), f32),
            pltpu.VMEM((CH, 2 * D), f32),
            pltpu.SemaphoreType.DMA,
            pltpu.SemaphoreType.DMA,
        ],
    )(idx2d, curv_all, z_gat, z_gin)

    wq, wk, wv = jnp.split(e2_in_W, 3, axis=0)
    bq, bk, bv = jnp.split(e2_in_b, 3, axis=0)

    def wcast(w):
        return w.astype(bf16)

    def b2d(b):
        return b.reshape(1, -1).astype(f32)

    weight_args = (
        wcast(e0_fc1_W), b2d(e0_fc1_b), wcast(e0_fc2_W), b2d(e0_fc2_b),
        wcast(e1_fc1_W), b2d(e1_fc1_b), wcast(e1_fc2_W), b2d(e1_fc2_b),
        wcast(wq), b2d(bq), wcast(wk), b2d(bk), wcast(wv), b2d(bv),
        wcast(e2_out_W), b2d(e2_out_b), wcast(e2_fc_W), b2d(e2_fc_b),
        wcast(e3_alpha_W), b2d(e3_alpha_b), wcast(e3_out_W), b2d(e3_out_b),
    )

    def wspec(w):
        return pl.BlockSpec(w.shape, lambda i: tuple(0 for _ in w.shape))

    out_sorted = pl.pallas_call(
        _expert_kernel,
        grid=(NBLK,),
        in_specs=[
            pl.BlockSpec(memory_space=pltpu.SMEM),
            pl.BlockSpec(memory_space=pltpu.SMEM),
            pl.BlockSpec((TE, 2 * D), lambda i: (i, 0)),
        ] + [wspec(w) for w in weight_args],
        out_specs=pl.BlockSpec((TE, D), lambda i: (i, 0)),
        out_shape=jax.ShapeDtypeStruct((NPAD, D), f32),
    )(block_expert, avg_prob, zs, *weight_args)

    output = pl.kernel(
        _sc_combine_kernel,
        out_type=jax.ShapeDtypeStruct((B, D), f32),
        mesh=mesh,
        compiler_params=pltpu.CompilerParams(needs_layout_passes=False),
        scratch_types=[
            pltpu.VMEM((ROWS_W, CH), i32),
            pltpu.VMEM((CH, D), f32),
            pltpu.VMEM((CH, D), f32),
            pltpu.SemaphoreType.DMA,
            pltpu.SemaphoreType.DMA,
        ],
    )(out_sorted, pos2d)

    return output, aux_loss
